# Initial kernel scaffold; baseline (speedup 1.0000x reference)
import jax
import jax.numpy as jnp
from jax.experimental import pallas as pl


def kernel(idx, table):
    raise NotImplementedError("write your pallas kernel here")


# speedup vs baseline: 1.4588x; 1.4588x over previous
import functools

import jax
import jax.numpy as jnp
from jax import lax
from jax.experimental import pallas as pl
from jax.experimental.pallas import tpu as pltpu
from jax.experimental.pallas import tpu_sc as plsc

_N_EMBD = 32
_B_TOTAL = 4096 * 200
_NW = 32
_B_PER_W = _B_TOTAL // _NW
_CHUNK = 1024
_N_CHUNKS = _B_PER_W // _CHUNK


_mesh = plsc.VectorSubcoreMesh(core_axis_name="c", subcore_axis_name="s")


@functools.partial(
    pl.kernel,
    mesh=_mesh,
    out_type=jax.ShapeDtypeStruct((_B_TOTAL, _N_EMBD), jnp.float32),
    scratch_types=[
        pltpu.VMEM((_CHUNK,), jnp.int32),
        pltpu.VMEM((_CHUNK, _N_EMBD), jnp.float32),
        pltpu.SemaphoreType.DMA,
    ],
    compiler_params=pltpu.CompilerParams(use_tc_tiling_on_sc=False),
)
def _gather_kernel(idx_hbm, table_hbm, out_hbm, idx_v, rows_v, sem):
    wid = lax.axis_index("s") * 2 + lax.axis_index("c")
    base = wid * _B_PER_W

    def body(i, carry):
        off = base + i * _CHUNK
        pltpu.sync_copy(idx_hbm.at[pl.ds(off, _CHUNK)], idx_v)
        pltpu.async_copy(table_hbm.at[idx_v], rows_v, sem).wait()
        pltpu.sync_copy(rows_v, out_hbm.at[pl.ds(off, _CHUNK)])
        return carry

    lax.fori_loop(0, _N_CHUNKS, body, 0)


def kernel(idx, table):
    flat = idx.reshape(-1)
    out = _gather_kernel(flat, table)
    return out.reshape(idx.shape + (_N_EMBD,))


# speedup vs baseline: 1.4914x; 1.0224x over previous
import functools

import jax
import jax.numpy as jnp
from jax import lax
from jax.experimental import pallas as pl
from jax.experimental.pallas import tpu as pltpu
from jax.experimental.pallas import tpu_sc as plsc

_N_EMBD = 32
_B_TOTAL = 4096 * 200
_NW = 32
_B_PER_W = _B_TOTAL // _NW
_CHUNK = 640
_NBUF = 4
_N_CHUNKS = _B_PER_W // _CHUNK
_NGRP = _N_CHUNKS // _NBUF


_mesh = plsc.VectorSubcoreMesh(core_axis_name="c", subcore_axis_name="s")


@functools.partial(
    pl.kernel,
    mesh=_mesh,
    out_type=jax.ShapeDtypeStruct((_B_TOTAL, _N_EMBD), jnp.float32),
    scratch_types=[
        pltpu.VMEM((_B_PER_W,), jnp.int32),
    ]
    + [pltpu.VMEM((_CHUNK, _N_EMBD), jnp.float32) for _ in range(_NBUF)]
    + [pltpu.SemaphoreType.DMA for _ in range(2 * _NBUF)],
    compiler_params=pltpu.CompilerParams(use_tc_tiling_on_sc=False),
)
def _gather_kernel(idx_hbm, table_hbm, out_hbm, idx_v, *bufs_and_sems):
    rows = bufs_and_sems[:_NBUF]
    gsem = bufs_and_sems[_NBUF:2 * _NBUF]
    wsem = bufs_and_sems[2 * _NBUF:]

    wid = lax.axis_index("s") * 2 + lax.axis_index("c")
    base = wid * _B_PER_W

    pltpu.sync_copy(idx_hbm.at[pl.ds(base, _B_PER_W)], idx_v)

    def gather_copy(i, b):
        return pltpu.make_async_copy(
            table_hbm.at[idx_v.at[pl.ds(i * _CHUNK, _CHUNK)]],
            rows[b],
            gsem[b],
        )

    def write_copy(i, b):
        return pltpu.make_async_copy(
            rows[b],
            out_hbm.at[pl.ds(base + i * _CHUNK, _CHUNK)],
            wsem[b],
        )

    for b in range(_NBUF):
        gather_copy(b, b).start()

    @pl.loop(0, _NGRP)
    def _grp(g):
        for b in range(_NBUF):
            i = g * _NBUF + b
            gather_copy(i, b).wait()
            write_copy(i, b).start()
        for b in range(_NBUF):
            i = g * _NBUF + b
            write_copy(i, b).wait()

            @pl.when(g < _NGRP - 1)
            def _():
                gather_copy(i + _NBUF, b).start()


def kernel(idx, table):
    flat = idx.reshape(-1)
    out = _gather_kernel(flat, table)
    return out.reshape(idx.shape + (_N_EMBD,))
